# initial kernel scaffold (unmeasured)
import jax
import jax.numpy as jnp
from jax import lax
from jax.experimental import pallas as pl
from jax.experimental.pallas import tpu as pltpu

N_DEV = 8


def kernel(x, w_mat):
    m_per, k = x.shape
    n = w_mat.shape[1]
    n_per = n // N_DEV

    def body(x_ref, w_ref, out_ref, blk_ref, send_sems, recv_sems):
        my_i = lax.axis_index("i")

        barrier_sem = pltpu.get_barrier_semaphore()
        for o in range(1, N_DEV):
            peer = (my_i + o) % N_DEV
            pl.semaphore_signal(
                barrier_sem, inc=1,
                device_id=(peer,), device_id_type=pl.DeviceIdType.MESH,
            )
        pl.semaphore_wait(barrier_sem, N_DEV - 1)

        y = jnp.maximum(
            jnp.dot(x_ref[:, :], w_ref[:, :], preferred_element_type=jnp.float32),
            0.0,
        )
        for j in range(N_DEV):
            blk_ref[j] = y[:, j * n_per:(j + 1) * n_per]

        own = lax.dynamic_slice(y, (0, my_i * n_per), (m_per, n_per))
        out_ref[pl.ds(my_i * m_per, m_per), :] = own

        rdmas = []
        for o in range(1, N_DEV):
            dst = (my_i + o) % N_DEV
            rdma = pltpu.make_async_remote_copy(
                src_ref=blk_ref.at[dst],
                dst_ref=out_ref.at[pl.ds(my_i * m_per, m_per)],
                send_sem=send_sems.at[o],
                recv_sem=recv_sems.at[o],
                device_id=(dst,),
                device_id_type=pl.DeviceIdType.MESH,
            )
            rdma.start()
            rdmas.append(rdma)

        for o in range(1, N_DEV):
            src_dev = (my_i - o + N_DEV) % N_DEV
            recv = pltpu.make_async_remote_copy(
                src_ref=blk_ref.at[0],
                dst_ref=out_ref.at[pl.ds(src_dev * m_per, m_per)],
                send_sem=send_sems.at[0],
                recv_sem=recv_sems.at[o],
                device_id=(src_dev,),
                device_id_type=pl.DeviceIdType.MESH,
            )
            recv.wait_recv()

        for rdma in rdmas:
            rdma.wait_send()

    return pl.pallas_call(
        body,
        out_shape=jax.ShapeDtypeStruct((N_DEV * m_per, n_per), jnp.float32),
        in_specs=[
            pl.BlockSpec(memory_space=pltpu.VMEM),
            pl.BlockSpec(memory_space=pltpu.VMEM),
        ],
        out_specs=pl.BlockSpec(memory_space=pltpu.VMEM),
        scratch_shapes=[
            pltpu.VMEM((N_DEV, m_per, n_per), jnp.float32),
            pltpu.SemaphoreType.DMA((N_DEV,)),
            pltpu.SemaphoreType.DMA((N_DEV,)),
        ],
        compiler_params=pltpu.CompilerParams(collective_id=0),
    )(x, w_mat)


# baseline (device time: 13781 ns/iter reference)
import jax
import jax.numpy as jnp
from jax import lax
from jax.experimental import pallas as pl
from jax.experimental.pallas import tpu as pltpu

N_DEV = 8


def kernel(x, w_mat):
    m_per, k = x.shape
    n = w_mat.shape[1]
    n_per = n // N_DEV

    def body(x_ref, w_ref, out_ref, blk_ref, send_sems, recv_sems):
        my_i = lax.axis_index("i")

        barrier_sem = pltpu.get_barrier_semaphore()
        for o in range(1, N_DEV):
            peer = (my_i + o) % N_DEV
            pl.semaphore_signal(
                barrier_sem, inc=1,
                device_id=(peer,), device_id_type=pl.DeviceIdType.MESH,
            )
        pl.semaphore_wait(barrier_sem, N_DEV - 1)

        y = jnp.maximum(
            jnp.dot(x_ref[:, :], w_ref[:, :], preferred_element_type=jnp.float32),
            0.0,
        )
        for j in range(N_DEV):
            blk_ref[j] = y[:, j * n_per:(j + 1) * n_per]

        out_ref[pl.ds(my_i * m_per, m_per), :] = blk_ref[my_i]

        rdmas = []
        for o in range(1, N_DEV):
            dst = (my_i + o) % N_DEV
            rdma = pltpu.make_async_remote_copy(
                src_ref=blk_ref.at[dst],
                dst_ref=out_ref.at[pl.ds(my_i * m_per, m_per)],
                send_sem=send_sems.at[o],
                recv_sem=recv_sems.at[o],
                device_id=(dst,),
                device_id_type=pl.DeviceIdType.MESH,
            )
            rdma.start()
            rdmas.append(rdma)

        for o in range(1, N_DEV):
            src_dev = (my_i - o + N_DEV) % N_DEV
            recv = pltpu.make_async_remote_copy(
                src_ref=blk_ref.at[0],
                dst_ref=out_ref.at[pl.ds(src_dev * m_per, m_per)],
                send_sem=send_sems.at[0],
                recv_sem=recv_sems.at[o],
                device_id=(src_dev,),
                device_id_type=pl.DeviceIdType.MESH,
            )
            recv.wait_recv()

        for rdma in rdmas:
            rdma.wait_send()

    return pl.pallas_call(
        body,
        out_shape=jax.ShapeDtypeStruct((N_DEV * m_per, n_per), jnp.float32),
        in_specs=[
            pl.BlockSpec(memory_space=pltpu.VMEM),
            pl.BlockSpec(memory_space=pltpu.VMEM),
        ],
        out_specs=pl.BlockSpec(memory_space=pltpu.VMEM),
        scratch_shapes=[
            pltpu.VMEM((N_DEV, m_per, n_per), jnp.float32),
            pltpu.SemaphoreType.DMA((N_DEV,)),
            pltpu.SemaphoreType.DMA((N_DEV,)),
        ],
        compiler_params=pltpu.CompilerParams(collective_id=0),
    )(x, w_mat)
